# baseline (speedup 1.0000x reference)
import functools

import jax
import jax.numpy as jnp
from jax import lax
from jax.experimental import pallas as pl
from jax.experimental.pallas import tpu as pltpu
from jax.experimental.pallas import tpu_sc as plsc

_N, _E, _D, _C = 10000, 320000, 128, 7
_NC, _NS = 2, 16
_NW = _NC * _NS
_RANGE = 313
_NPAD = _NW * _RANGE
_CHUNK = _E // _NW
_REG = _CHUNK + 16 * _NW
_SORTPAD = 4096
_WSEG = 4096
_G = 128
_MAGIC, _SHIFT = 13401, 22


def _bin_body(src_hbm, dst_hbm, sorted_hbm, starts_hbm, counts_hbm,
              src_c, dst_c, sorted_c, starts_v, counts_v, cnt_sm, cur_sm):
    c = lax.axis_index("c")
    s = lax.axis_index("s")
    wid = s * _NC + c

    pltpu.sync_copy(src_hbm.at[pl.ds(wid * _CHUNK, _CHUNK)], src_c)
    pltpu.sync_copy(dst_hbm.at[pl.ds(wid * _CHUNK, _CHUNK)], dst_c)

    def zero_sm(o, carry):
        cnt_sm[o] = 0
        return carry

    lax.fori_loop(0, _NW, zero_sm, 0)

    def hist_body(i, carry):
        d16 = dst_c[pl.ds(i * 16, 16)]
        o16 = (d16 * _MAGIC) >> _SHIFT
        for jj in range(16):
            o = o16[jj]
            cnt_sm[o] = cnt_sm[o] + 1
        return carry

    lax.fori_loop(0, _CHUNK // 16, hist_body, 0)

    def pref_body(o, run):
        cval = cnt_sm[o]
        local = run + 16 * o
        cur_sm[o] = local
        starts_v[pl.ds(o, 16)] = jnp.full((16,), wid * _REG, jnp.int32) + local
        counts_v[pl.ds(o, 16)] = jnp.full((16,), 0, jnp.int32) + cval
        return run + cval

    lax.fori_loop(0, _NW, pref_body, 0)

    def place_body(i, carry):
        d16 = dst_c[pl.ds(i * 16, 16)]
        s16 = src_c[pl.ds(i * 16, 16)]
        o16 = (d16 * _MAGIC) >> _SHIFT
        du16 = d16 - o16 * _RANGE
        p16 = (s16 << 9) | du16
        for jj in range(16):
            o = o16[jj]
            p = p16[jj]
            cpos = cur_sm[o]
            cur_sm[o] = cpos + 1
            sorted_c[pl.ds(cpos, 16)] = jnp.full((16,), 0, jnp.int32) + p
        return carry

    lax.fori_loop(0, _CHUNK // 16, place_body, 0)

    pltpu.sync_copy(sorted_c.at[pl.ds(0, _REG)],
                    sorted_hbm.at[pl.ds(wid * _REG, _REG)])
    pltpu.sync_copy(starts_v.at[pl.ds(0, _NW)],
                    starts_hbm.at[pl.ds(wid * _NW, _NW)])
    pltpu.sync_copy(counts_v.at[pl.ds(0, _NW)],
                    counts_hbm.at[pl.ds(wid * _NW, _NW)])


def _agg_body(x_hbm, sorted_hbm, starts_hbm, counts_hbm, out_hbm,
              tb_s, tb_c, segbuf, src_sel, dst_sel, rows0, rows1, acc,
              sem0, sem1):
    c = lax.axis_index("c")
    s = lax.axis_index("s")
    wid = s * _NC + c

    neg16 = jnp.full((16,), -jnp.inf, dtype=jnp.float32)
    zero16 = jnp.zeros((16,), dtype=jnp.int32)
    trash16 = jnp.full((16,), _RANGE, dtype=jnp.int32)
    lanes = lax.iota(jnp.int32, 16)

    pltpu.sync_copy(starts_hbm, tb_s)
    pltpu.sync_copy(counts_hbm, tb_c)

    def init_body(i, carry):
        acc[pl.ds(i * 16, 16)] = neg16
        return carry

    lax.fori_loop(0, ((_RANGE + 1) * _D) // 16, init_body, 0)

    def rmw(rows, cbase):
        def grp_body(g, carry3):
            d16v = dst_sel[pl.ds(cbase + g * 16, 16)]
            offv = d16v * _D
            dls = [offv[jj] for jj in range(16)]
            for jj in range(16):
                abase = dls[jj]
                r = g * 16 + jj
                rv = [rows[r, f * 16:(f + 1) * 16] for f in range(8)]
                av = [acc[pl.ds(abase + f * 16, 16)] for f in range(8)]
                mx = [jnp.maximum(av[f], rv[f]) for f in range(8)]
                for f in range(8):
                    acc[pl.ds(abase + f * 16, 16)] = mx[f]
            return carry3

        lax.fori_loop(0, _G // 16, grp_body, 0)

    def flush(cnt):
        for t in range(_G // 16):
            src_sel[pl.ds(cnt + t * 16, 16)] = zero16
            dst_sel[pl.ds(cnt + t * 16, 16)] = trash16
        nch = (cnt + _G - 1) // _G

        @pl.when(nch > 0)
        def _():
            pltpu.async_copy(x_hbm.at[src_sel.at[pl.ds(0, _G)]], rows0, sem0)

        def step(cur, csem, nxt, nsem, ci):
            @pl.when(ci + 1 < nch)
            def _():
                pltpu.async_copy(
                    x_hbm.at[src_sel.at[pl.ds((ci + 1) * _G, _G)]], nxt, nsem)
            pltpu.make_async_copy(
                x_hbm.at[src_sel.at[pl.ds(ci * _G, _G)]], cur, csem).wait()
            rmw(cur, ci * _G)

        def chunk_body(ci, carry2):
            @pl.when(ci % 2 == 0)
            def _():
                step(rows0, sem0, rows1, sem1, ci)

            @pl.when(ci % 2 == 1)
            def _():
                step(rows1, sem1, rows0, sem0, ci)
            return carry2

        lax.fori_loop(0, nch, chunk_body, 0)

    def seg_body(t, carry):
        p = t * _NW + wid
        grp_s = tb_s[pl.ds((p >> 4) << 4, 16)]
        grp_c = tb_c[pl.ds((p >> 4) << 4, 16)]
        lane = jnp.full((16,), p & 15, jnp.int32)
        start = grp_s.at[lane].get(mode="promise_in_bounds")[0]
        count = grp_c.at[lane].get(mode="promise_in_bounds")[0]
        a0 = pl.multiple_of((start >> 3) << 3, 8)
        skip = start - a0
        total = skip + count
        nwin = (total + _WSEG - 1) // _WSEG

        def win_body(w, carry2):
            wbase = a0 + w * _WSEG
            pltpu.sync_copy(sorted_hbm.at[pl.ds(wbase, _WSEG)], segbuf)
            rem = total - w * _WSEG
            ngrp = (jnp.minimum(rem, _WSEG) + 15) // 16
            startv = jnp.full((16,), start, jnp.int32)
            endv = jnp.full((16,), start + count, jnp.int32)

            def grp_body(g, cnt):
                vpos = jnp.full((16,), wbase + g * 16, jnp.int32) + lanes
                mv = (vpos >= startv) & (vpos < endv)
                p16 = segbuf[pl.ds(g * 16, 16)]
                s16 = (p16 >> 9) & jnp.full((16,), 16383, jnp.int32)
                du16 = p16 & jnp.full((16,), 511, jnp.int32)
                src_sel[pl.ds(cnt, 16)] = jnp.where(mv, s16, zero16)
                dst_sel[pl.ds(cnt, 16)] = jnp.where(mv, du16, trash16)
                return cnt + 16

            cnt = lax.fori_loop(0, ngrp, grp_body, 0)
            flush(cnt)
            return carry2

        lax.fori_loop(0, nwin, win_body, 0)
        return carry

    lax.fori_loop(0, _NW, seg_body, 0)

    pltpu.sync_copy(acc.at[pl.ds(0, _RANGE * _D)],
                    out_hbm.at[pl.ds(wid * _RANGE * _D, _RANGE * _D)])


def _segment_max_sc(x, src, dst):
    mesh = plsc.VectorSubcoreMesh(core_axis_name="c", subcore_axis_name="s")
    cp = pltpu.CompilerParams(needs_layout_passes=False)
    binf = pl.kernel(
        _bin_body,
        out_type=(
            jax.ShapeDtypeStruct((_NW * _REG + _SORTPAD,), jnp.int32),
            jax.ShapeDtypeStruct((_NW * _NW,), jnp.int32),
            jax.ShapeDtypeStruct((_NW * _NW,), jnp.int32),
        ),
        mesh=mesh,
        compiler_params=cp,
        scratch_types=[
            pltpu.VMEM((_CHUNK,), jnp.int32),
            pltpu.VMEM((_CHUNK,), jnp.int32),
            pltpu.VMEM((_REG + 16,), jnp.int32),
            pltpu.VMEM((_NW + 16,), jnp.int32),
            pltpu.VMEM((_NW + 16,), jnp.int32),
            pltpu.SMEM((_NW,), jnp.int32),
            pltpu.SMEM((_NW,), jnp.int32),
        ],
    )
    sorted_a, starts_a, counts_a = binf(src, dst)
    aggf = pl.kernel(
        _agg_body,
        out_type=jax.ShapeDtypeStruct((_NPAD * _D,), jnp.float32),
        mesh=mesh,
        compiler_params=cp,
        scratch_types=[
            pltpu.VMEM((_NW * _NW,), jnp.int32),
            pltpu.VMEM((_NW * _NW,), jnp.int32),
            pltpu.VMEM((_WSEG,), jnp.int32),
            pltpu.VMEM((_WSEG + _G + 16,), jnp.int32),
            pltpu.VMEM((_WSEG + _G + 16,), jnp.int32),
            pltpu.VMEM((_G, _D), jnp.float32),
            pltpu.VMEM((_G, _D), jnp.float32),
            pltpu.VMEM(((_RANGE + 1) * _D,), jnp.float32),
            pltpu.SemaphoreType.DMA,
            pltpu.SemaphoreType.DMA,
        ],
    )
    return aggf(x, sorted_a, starts_a, counts_a)


def _tc_body(x_ref, agg_ref, wl_ref, b_ref, wr_ref, o_ref):
    a = agg_ref[...]
    a = jnp.where(a == -jnp.inf, 0.0, a)
    z = (jnp.dot(a, wl_ref[...], preferred_element_type=jnp.float32)
         + jnp.dot(x_ref[...], wr_ref[...], preferred_element_type=jnp.float32)
         + b_ref[...])
    col = lax.broadcasted_iota(jnp.int32, z.shape, 1)
    z = jnp.where(col < _C, z, -jnp.inf)
    m = jnp.max(z, axis=1, keepdims=True)
    zs = z - m
    lse = jnp.log(jnp.sum(jnp.exp(zs), axis=1, keepdims=True))
    o_ref[...] = zs - lse


_BN = 400


def _dense_tc(x, agg, wl, b, wr):
    return pl.pallas_call(
        _tc_body,
        grid=(_N // _BN,),
        in_specs=[
            pl.BlockSpec((_BN, _D), lambda i: (i, 0)),
            pl.BlockSpec((_BN, _D), lambda i: (i, 0)),
            pl.BlockSpec((_D, 128), lambda i: (0, 0)),
            pl.BlockSpec((1, 128), lambda i: (0, 0)),
            pl.BlockSpec((_D, 128), lambda i: (0, 0)),
        ],
        out_specs=pl.BlockSpec((_BN, 128), lambda i: (i, 0)),
        out_shape=jax.ShapeDtypeStruct((_N, 128), jnp.float32),
    )(x, agg, wl, b, wr)


def kernel(x, edge_index, W_l, b_l, W_r):
    src = edge_index[0]
    dst = edge_index[1]
    aggf = _segment_max_sc(x, src, dst)
    agg = aggf.reshape(_NPAD, _D)
    wl = jnp.zeros((_D, 128), jnp.float32).at[:, :_C].set(W_l.T)
    wr = jnp.zeros((_D, 128), jnp.float32).at[:, :_C].set(W_r.T)
    b = jnp.zeros((1, 128), jnp.float32).at[0, :_C].set(b_l)
    out = _dense_tc(x, agg, wl, b, wr)
    return out[:, :_C]


# speedup vs baseline: 1.0027x; 1.0027x over previous
import functools

import jax
import jax.numpy as jnp
from jax import lax
from jax.experimental import pallas as pl
from jax.experimental.pallas import tpu as pltpu
from jax.experimental.pallas import tpu_sc as plsc

_N, _E, _D, _C = 10000, 320000, 128, 7
_NC, _NS = 2, 16
_NW = _NC * _NS
_RANGE = 313
_NPAD = _NW * _RANGE
_CHUNK = _E // _NW
_REG = _CHUNK + 16 * _NW
_SORTPAD = 4096
_WSEG = 4096
_G = 128
_NSTREAM = 4
_MAGIC, _SHIFT = 13401, 22


def _bin_body(src_hbm, dst_hbm, sorted_hbm, starts_hbm, counts_hbm,
              src_c, dst_c, sorted_c, starts_v, counts_v, cnt_sm, cur_sm):
    c = lax.axis_index("c")
    s = lax.axis_index("s")
    wid = s * _NC + c

    pltpu.sync_copy(src_hbm.at[pl.ds(wid * _CHUNK, _CHUNK)], src_c)
    pltpu.sync_copy(dst_hbm.at[pl.ds(wid * _CHUNK, _CHUNK)], dst_c)

    def zero_sm(o, carry):
        cnt_sm[o] = 0
        return carry

    lax.fori_loop(0, _NW, zero_sm, 0)

    def hist_body(i, carry):
        d16 = dst_c[pl.ds(i * 16, 16)]
        o16 = (d16 * _MAGIC) >> _SHIFT
        for jj in range(16):
            o = o16[jj]
            cnt_sm[o] = cnt_sm[o] + 1
        return carry

    lax.fori_loop(0, _CHUNK // 16, hist_body, 0)

    def pref_body(o, run):
        cval = cnt_sm[o]
        local = run + 16 * o
        cur_sm[o] = local
        starts_v[pl.ds(o, 16)] = jnp.full((16,), wid * _REG, jnp.int32) + local
        counts_v[pl.ds(o, 16)] = jnp.full((16,), 0, jnp.int32) + cval
        return run + cval

    lax.fori_loop(0, _NW, pref_body, 0)

    def place_body(i, carry):
        d16 = dst_c[pl.ds(i * 16, 16)]
        s16 = src_c[pl.ds(i * 16, 16)]
        o16 = (d16 * _MAGIC) >> _SHIFT
        du16 = d16 - o16 * _RANGE
        p16 = (s16 << 9) | du16
        for jj in range(16):
            o = o16[jj]
            p = p16[jj]
            cpos = cur_sm[o]
            cur_sm[o] = cpos + 1
            sorted_c[pl.ds(cpos, 16)] = jnp.full((16,), 0, jnp.int32) + p
        return carry

    lax.fori_loop(0, _CHUNK // 16, place_body, 0)

    pltpu.sync_copy(sorted_c.at[pl.ds(0, _REG)],
                    sorted_hbm.at[pl.ds(wid * _REG, _REG)])
    pltpu.sync_copy(starts_v.at[pl.ds(0, _NW)],
                    starts_hbm.at[pl.ds(wid * _NW, _NW)])
    pltpu.sync_copy(counts_v.at[pl.ds(0, _NW)],
                    counts_hbm.at[pl.ds(wid * _NW, _NW)])


def _agg_body(x_hbm, sorted_hbm, starts_hbm, counts_hbm, out_hbm,
              tb_s, tb_c, segbuf, src_sel, dst_sel, rows0, rows1, acc,
              *sems):
    c = lax.axis_index("c")
    s = lax.axis_index("s")
    wid = s * _NC + c

    neg16 = jnp.full((16,), -jnp.inf, dtype=jnp.float32)
    zero16 = jnp.zeros((16,), dtype=jnp.int32)
    trash16 = jnp.full((16,), _RANGE, dtype=jnp.int32)
    lanes = lax.iota(jnp.int32, 16)

    pltpu.sync_copy(starts_hbm, tb_s)
    pltpu.sync_copy(counts_hbm, tb_c)

    def init_body(i, carry):
        acc[pl.ds(i * 16, 16)] = neg16
        return carry

    lax.fori_loop(0, ((_RANGE + 1) * _D) // 16, init_body, 0)

    def rmw(rows, cbase):
        def grp_body(g, carry3):
            d16v = dst_sel[pl.ds(cbase + g * 16, 16)]
            offv = d16v * _D
            dls = [offv[jj] for jj in range(16)]
            for jj in range(16):
                abase = dls[jj]
                r = g * 16 + jj
                rv = [rows[r, f * 16:(f + 1) * 16] for f in range(8)]
                av = [acc[pl.ds(abase + f * 16, 16)] for f in range(8)]
                mx = [jnp.maximum(av[f], rv[f]) for f in range(8)]
                for f in range(8):
                    acc[pl.ds(abase + f * 16, 16)] = mx[f]
            return carry3

        lax.fori_loop(0, _G // 16, grp_body, 0)

    def flush(cnt):
        for t in range(_G // 16):
            src_sel[pl.ds(cnt + t * 16, 16)] = zero16
            dst_sel[pl.ds(cnt + t * 16, 16)] = trash16
        nch = (cnt + _G - 1) // _G
        _Q = _G // _NSTREAM

        def start_gathers(buf, bsems, cbase):
            for q in range(_NSTREAM):
                pltpu.async_copy(
                    x_hbm.at[src_sel.at[pl.ds(cbase + q * _Q, _Q)]],
                    buf.at[pl.ds(q * _Q, _Q)], bsems[q])

        def wait_gathers(buf, bsems, cbase):
            for q in range(_NSTREAM):
                pltpu.make_async_copy(
                    x_hbm.at[src_sel.at[pl.ds(cbase + q * _Q, _Q)]],
                    buf.at[pl.ds(q * _Q, _Q)], bsems[q]).wait()

        semA = sems[:_NSTREAM]
        semB = sems[_NSTREAM:]

        @pl.when(nch > 0)
        def _():
            start_gathers(rows0, semA, 0)

        def step(cur, csems, nxt, nsems, ci):
            @pl.when(ci + 1 < nch)
            def _():
                start_gathers(nxt, nsems, (ci + 1) * _G)
            wait_gathers(cur, csems, ci * _G)
            rmw(cur, ci * _G)

        def chunk_body(ci, carry2):
            @pl.when(ci % 2 == 0)
            def _():
                step(rows0, semA, rows1, semB, ci)

            @pl.when(ci % 2 == 1)
            def _():
                step(rows1, semB, rows0, semA, ci)
            return carry2

        lax.fori_loop(0, nch, chunk_body, 0)

    def seg_body(t, carry):
        p = t * _NW + wid
        grp_s = tb_s[pl.ds((p >> 4) << 4, 16)]
        grp_c = tb_c[pl.ds((p >> 4) << 4, 16)]
        lane = jnp.full((16,), p & 15, jnp.int32)
        start = grp_s.at[lane].get(mode="promise_in_bounds")[0]
        count = grp_c.at[lane].get(mode="promise_in_bounds")[0]
        a0 = pl.multiple_of((start >> 3) << 3, 8)
        skip = start - a0
        total = skip + count
        nwin = (total + _WSEG - 1) // _WSEG

        def win_body(w, carry2):
            wbase = a0 + w * _WSEG
            pltpu.sync_copy(sorted_hbm.at[pl.ds(wbase, _WSEG)], segbuf)
            rem = total - w * _WSEG
            ngrp = (jnp.minimum(rem, _WSEG) + 15) // 16
            startv = jnp.full((16,), start, jnp.int32)
            endv = jnp.full((16,), start + count, jnp.int32)

            def grp_body(g, cnt):
                vpos = jnp.full((16,), wbase + g * 16, jnp.int32) + lanes
                mv = (vpos >= startv) & (vpos < endv)
                p16 = segbuf[pl.ds(g * 16, 16)]
                s16 = (p16 >> 9) & jnp.full((16,), 16383, jnp.int32)
                du16 = p16 & jnp.full((16,), 511, jnp.int32)
                src_sel[pl.ds(cnt, 16)] = jnp.where(mv, s16, zero16)
                dst_sel[pl.ds(cnt, 16)] = jnp.where(mv, du16, trash16)
                return cnt + 16

            cnt = lax.fori_loop(0, ngrp, grp_body, 0)
            flush(cnt)
            return carry2

        lax.fori_loop(0, nwin, win_body, 0)
        return carry

    lax.fori_loop(0, _NW, seg_body, 0)

    pltpu.sync_copy(acc.at[pl.ds(0, _RANGE * _D)],
                    out_hbm.at[pl.ds(wid * _RANGE * _D, _RANGE * _D)])


def _segment_max_sc(x, src, dst):
    mesh = plsc.VectorSubcoreMesh(core_axis_name="c", subcore_axis_name="s")
    cp = pltpu.CompilerParams(needs_layout_passes=False)
    binf = pl.kernel(
        _bin_body,
        out_type=(
            jax.ShapeDtypeStruct((_NW * _REG + _SORTPAD,), jnp.int32),
            jax.ShapeDtypeStruct((_NW * _NW,), jnp.int32),
            jax.ShapeDtypeStruct((_NW * _NW,), jnp.int32),
        ),
        mesh=mesh,
        compiler_params=cp,
        scratch_types=[
            pltpu.VMEM((_CHUNK,), jnp.int32),
            pltpu.VMEM((_CHUNK,), jnp.int32),
            pltpu.VMEM((_REG + 16,), jnp.int32),
            pltpu.VMEM((_NW + 16,), jnp.int32),
            pltpu.VMEM((_NW + 16,), jnp.int32),
            pltpu.SMEM((_NW,), jnp.int32),
            pltpu.SMEM((_NW,), jnp.int32),
        ],
    )
    sorted_a, starts_a, counts_a = binf(src, dst)
    aggf = pl.kernel(
        _agg_body,
        out_type=jax.ShapeDtypeStruct((_NPAD * _D,), jnp.float32),
        mesh=mesh,
        compiler_params=cp,
        scratch_types=[
            pltpu.VMEM((_NW * _NW,), jnp.int32),
            pltpu.VMEM((_NW * _NW,), jnp.int32),
            pltpu.VMEM((_WSEG,), jnp.int32),
            pltpu.VMEM((_WSEG + _G + 16,), jnp.int32),
            pltpu.VMEM((_WSEG + _G + 16,), jnp.int32),
            pltpu.VMEM((_G, _D), jnp.float32),
            pltpu.VMEM((_G, _D), jnp.float32),
            pltpu.VMEM(((_RANGE + 1) * _D,), jnp.float32),
        ] + [pltpu.SemaphoreType.DMA] * (2 * _NSTREAM),
    )
    return aggf(x, sorted_a, starts_a, counts_a)


def _tc_body(x_ref, agg_ref, wl_ref, b_ref, wr_ref, o_ref):
    a = agg_ref[...]
    a = jnp.where(a == -jnp.inf, 0.0, a)
    z = (jnp.dot(a, wl_ref[...], preferred_element_type=jnp.float32)
         + jnp.dot(x_ref[...], wr_ref[...], preferred_element_type=jnp.float32)
         + b_ref[...])
    col = lax.broadcasted_iota(jnp.int32, z.shape, 1)
    z = jnp.where(col < _C, z, -jnp.inf)
    m = jnp.max(z, axis=1, keepdims=True)
    zs = z - m
    lse = jnp.log(jnp.sum(jnp.exp(zs), axis=1, keepdims=True))
    o_ref[...] = zs - lse


_BN = 400


def _dense_tc(x, agg, wl, b, wr):
    return pl.pallas_call(
        _tc_body,
        grid=(_N // _BN,),
        in_specs=[
            pl.BlockSpec((_BN, _D), lambda i: (i, 0)),
            pl.BlockSpec((_BN, _D), lambda i: (i, 0)),
            pl.BlockSpec((_D, 128), lambda i: (0, 0)),
            pl.BlockSpec((1, 128), lambda i: (0, 0)),
            pl.BlockSpec((_D, 128), lambda i: (0, 0)),
        ],
        out_specs=pl.BlockSpec((_BN, 128), lambda i: (i, 0)),
        out_shape=jax.ShapeDtypeStruct((_N, 128), jnp.float32),
    )(x, agg, wl, b, wr)


def kernel(x, edge_index, W_l, b_l, W_r):
    src = edge_index[0]
    dst = edge_index[1]
    aggf = _segment_max_sc(x, src, dst)
    agg = aggf.reshape(_NPAD, _D)
    wl = jnp.zeros((_D, 128), jnp.float32).at[:, :_C].set(W_l.T)
    wr = jnp.zeros((_D, 128), jnp.float32).at[:, :_C].set(W_r.T)
    b = jnp.zeros((1, 128), jnp.float32).at[0, :_C].set(b_l)
    out = _dense_tc(x, agg, wl, b, wr)
    return out[:, :_C]


# speedup vs baseline: 8.0946x; 8.0724x over previous
import functools

import jax
import jax.numpy as jnp
from jax import lax
from jax.experimental import pallas as pl
from jax.experimental.pallas import tpu as pltpu
from jax.experimental.pallas import tpu_sc as plsc

_N, _E, _D, _C = 10000, 320000, 128, 7
_NC, _NS = 2, 16
_NW = _NC * _NS
_RANGE = 313
_NPAD = _NW * _RANGE
_D2 = _D // 2
_CHUNK = _E // _NW
_NKEY = _NW * _NW
_REG = _CHUNK + 16 * _NKEY
_SEGW = 2048
_SEGBUF = 12304
_SLOTCAP = 384
_SLOT = _SLOTCAP + 32
_META = _NW * _SLOT
_XROWS = 320
_MAGIC, _SHIFT = 13401, 22


def _bin_body(src_hbm, dst_hbm, sorted_hbm, starts_hbm, counts_hbm,
              src_c, dst_c, sorted_c, starts_v, counts_v, cnt_sm):
    c = lax.axis_index("c")
    s = lax.axis_index("s")
    wid = s * _NC + c

    pltpu.sync_copy(src_hbm.at[pl.ds(wid * _CHUNK, _CHUNK)], src_c)
    pltpu.sync_copy(dst_hbm.at[pl.ds(wid * _CHUNK, _CHUNK)], dst_c)

    def zero_sm(o, carry):
        cnt_sm[o] = 0
        return carry

    lax.fori_loop(0, _NKEY, zero_sm, 0)

    def hist_body(i, carry):
        d16 = dst_c[pl.ds(i * 16, 16)]
        s16 = src_c[pl.ds(i * 16, 16)]
        k16 = ((d16 * _MAGIC) >> _SHIFT) * _NW + ((s16 * _MAGIC) >> _SHIFT)
        for jj in range(16):
            kk = k16[jj]
            cnt_sm[kk] = cnt_sm[kk] + 1
        return carry

    lax.fori_loop(0, _CHUNK // 16, hist_body, 0)

    def pref_body(q, run):
        cval = cnt_sm[q]
        local = run + 16 * q
        cnt_sm[q] = local
        starts_v[pl.ds(q, 16)] = jnp.full((16,), wid * _REG, jnp.int32) + local
        counts_v[pl.ds(q, 16)] = jnp.full((16,), 0, jnp.int32) + cval
        return run + cval

    lax.fori_loop(0, _NKEY, pref_body, 0)

    def place_body(i, carry):
        d16 = dst_c[pl.ds(i * 16, 16)]
        s16 = src_c[pl.ds(i * 16, 16)]
        o16 = (d16 * _MAGIC) >> _SHIFT
        k16 = o16 * _NW + ((s16 * _MAGIC) >> _SHIFT)
        p16 = (s16 << 9) | (d16 - o16 * _RANGE)
        for jj in range(16):
            kk = k16[jj]
            p = p16[jj]
            cpos = cnt_sm[kk]
            cnt_sm[kk] = cpos + 1
            sorted_c[pl.ds(cpos, 16)] = jnp.full((16,), 0, jnp.int32) + p
        return carry

    lax.fori_loop(0, _CHUNK // 16, place_body, 0)

    pltpu.sync_copy(sorted_c.at[pl.ds(0, _REG)],
                    sorted_hbm.at[pl.ds(wid * _REG, _REG)])
    pltpu.sync_copy(starts_v.at[pl.ds(0, _NKEY)],
                    starts_hbm.at[pl.ds(wid * _NKEY, _NKEY)])
    pltpu.sync_copy(counts_v.at[pl.ds(0, _NKEY)],
                    counts_hbm.at[pl.ds(wid * _NKEY, _NKEY)])


def _agg_body(x_hbm, sorted_hbm, starts_hbm, counts_hbm, out_hbm,
              tb_s, tb_c, segbuf, meta, xblk0, xblk1, acc,
              cur_sm, semx0, semx1):
    c = lax.axis_index("c")
    s = lax.axis_index("s")
    wid = s * _NC + c

    neg32 = jnp.full((32,), -jnp.inf, dtype=jnp.bfloat16)
    neg16p = plsc.bitcast(neg32, jnp.float32)

    def init_body(i, carry):
        acc[pl.ds(i * 16, 16)] = neg16p
        return carry

    lax.fori_loop(0, ((_RANGE + 1) * _D2) // 16, init_body, 0)

    for cc in range(_NW):
        pltpu.sync_copy(starts_hbm.at[pl.ds(cc * _NKEY + wid * _NW, _NW)],
                        tb_s.at[pl.ds(cc * _NW, _NW)])
        pltpu.sync_copy(counts_hbm.at[pl.ds(cc * _NKEY + wid * _NW, _NW)],
                        tb_c.at[pl.ds(cc * _NW, _NW)])

    def reset_cur(b, carry):
        cur_sm[b] = b * _SLOT
        return carry

    lax.fori_loop(0, _NW, reset_cur, 0)

    def flush():
        def seal_b(b, carry):
            safe = ((b * _RANGE) << 9) | _RANGE
            meta[pl.ds(cur_sm[b], 16)] = jnp.full((16,), 0, jnp.int32) + safe
            return carry

        lax.fori_loop(0, _NW, seal_b, 0)

        pltpu.async_copy(x_hbm.at[pl.ds(0, _XROWS)],
                         xblk0.at[pl.ds(0, _XROWS)], semx0)

        def run_b(b, cst, xblk):
            rtot = cur_sm[b] - b * _SLOT
            ngrp = (rtot + 15) // 16
            base = cst

            def grp_body(g, carry):
                p16 = meta[pl.ds(b * _SLOT + g * 16, 16)]
                for jj in range(16):
                    p = p16[jj]
                    sloc = (p >> 9) - base
                    abase = (p & 511) * _D2
                    rv = [plsc.bitcast(xblk[sloc, f * 16:(f + 1) * 16],
                                       jnp.bfloat16) for f in range(4)]
                    av = [plsc.bitcast(acc[pl.ds(abase + f * 16, 16)],
                                       jnp.bfloat16) for f in range(4)]
                    mx = [plsc.bitcast(jnp.maximum(av[f], rv[f]), jnp.float32)
                          for f in range(4)]
                    for f in range(4):
                        acc[pl.ds(abase + f * 16, 16)] = mx[f]
                return carry

            lax.fori_loop(0, ngrp, grp_body, 0)

        def stepx(b, xb, sem, nxb, nsem):
            @pl.when(b + 1 < _NW)
            def _():
                nst = pl.multiple_of((((b + 1) * _RANGE) >> 3) << 3, 8)
                pltpu.async_copy(x_hbm.at[pl.ds(nst, _XROWS)],
                                 nxb.at[pl.ds(0, _XROWS)], nsem)
            cst = pl.multiple_of(((b * _RANGE) >> 3) << 3, 8)
            pltpu.make_async_copy(x_hbm.at[pl.ds(cst, _XROWS)],
                                  xb.at[pl.ds(0, _XROWS)], sem).wait()
            run_b(b, cst, xb)

        def blk_body(b, carry):
            @pl.when(b % 2 == 0)
            def _():
                stepx(b, xblk0, semx0, xblk1, semx1)

            @pl.when(b % 2 == 1)
            def _():
                stepx(b, xblk1, semx1, xblk0, semx0)
            return carry

        lax.fori_loop(0, _NW, blk_body, 0)
        lax.fori_loop(0, _NW, reset_cur, 0)

    def chunk_loop(cc, carry0):
        g0 = tb_s[pl.ds(cc * _NW, 16)]
        g1 = tb_s[pl.ds(cc * _NW + 16, 16)]
        gc1 = tb_c[pl.ds(cc * _NW + 16, 16)]
        seg0 = g0[0]
        segend = g1[15] + gc1[15]
        a0 = pl.multiple_of((seg0 >> 3) << 3, 8)
        span = segend - a0

        need = 0
        for half in range(2):
            cv = tb_c[pl.ds(cc * _NW + half * 16, 16)]
            for jj in range(16):
                b = half * 16 + jj
                over = jnp.where(
                    cur_sm[b] + cv[jj] > b * _SLOT + _SLOTCAP, 1, 0)
                need = need + over

        @pl.when(need > 0)
        def _():
            flush()

        def win_body(w, carry):
            pltpu.sync_copy(sorted_hbm.at[pl.ds(a0 + w * _SEGW, _SEGW)],
                            segbuf.at[pl.ds(w * _SEGW, _SEGW)])
            return carry

        lax.fori_loop(0, (span + _SEGW - 1) // _SEGW, win_body, 0)

        for half in range(2):
            sv = tb_s[pl.ds(cc * _NW + half * 16, 16)]
            cv = tb_c[pl.ds(cc * _NW + half * 16, 16)]
            for jj in range(16):
                b = half * 16 + jj
                sstart = sv[jj] - a0
                scount = cv[jj]
                cpos = cur_sm[b]
                ng = (scount + 15) // 16

                def cp_body(g, carry):
                    meta[pl.ds(cpos + g * 16, 16)] = (
                        segbuf[pl.ds(sstart + g * 16, 16)])
                    return carry

                lax.fori_loop(0, ng, cp_body, 0)
                cur_sm[b] = cpos + scount
        return carry0

    lax.fori_loop(0, _NW, chunk_loop, 0)
    flush()

    pltpu.sync_copy(acc.at[pl.ds(0, _RANGE * _D2)],
                    out_hbm.at[pl.ds(wid * _RANGE * _D2, _RANGE * _D2)])


def _segment_max_sc(x, src, dst):
    mesh = plsc.VectorSubcoreMesh(core_axis_name="c", subcore_axis_name="s")
    cp = pltpu.CompilerParams(needs_layout_passes=False)
    binf = pl.kernel(
        _bin_body,
        out_type=(
            jax.ShapeDtypeStruct((_NW * _REG + 2 * _SEGW,), jnp.int32),
            jax.ShapeDtypeStruct((_NW * _NKEY,), jnp.int32),
            jax.ShapeDtypeStruct((_NW * _NKEY,), jnp.int32),
        ),
        mesh=mesh,
        compiler_params=cp,
        scratch_types=[
            pltpu.VMEM((_CHUNK,), jnp.int32),
            pltpu.VMEM((_CHUNK,), jnp.int32),
            pltpu.VMEM((_REG + 16,), jnp.int32),
            pltpu.VMEM((_NKEY + 16,), jnp.int32),
            pltpu.VMEM((_NKEY + 16,), jnp.int32),
            pltpu.SMEM((_NKEY,), jnp.int32),
        ],
    )
    sorted_a, starts_a, counts_a = binf(src, dst)
    xp = jax.lax.bitcast_convert_type(
        jnp.pad(x, ((0, _NPAD + _XROWS - _RANGE - _N), (0, 0)))
        .astype(jnp.bfloat16)
        .reshape(_NPAD + _XROWS - _RANGE, _D2, 2), jnp.float32)
    aggf = pl.kernel(
        _agg_body,
        out_type=jax.ShapeDtypeStruct((_NPAD * _D2,), jnp.float32),
        mesh=mesh,
        compiler_params=cp,
        scratch_types=[
            pltpu.VMEM((_NKEY + 16,), jnp.int32),
            pltpu.VMEM((_NKEY + 16,), jnp.int32),
            pltpu.VMEM((_SEGBUF,), jnp.int32),
            pltpu.VMEM((_META,), jnp.int32),
            pltpu.VMEM((_XROWS, _D2), jnp.float32),
            pltpu.VMEM((_XROWS, _D2), jnp.float32),
            pltpu.VMEM(((_RANGE + 1) * _D2,), jnp.float32),
            pltpu.SMEM((_NW,), jnp.int32),
            pltpu.SemaphoreType.DMA,
            pltpu.SemaphoreType.DMA,
        ],
    )
    return aggf(xp, sorted_a, starts_a, counts_a)


def _tc_body(x_ref, agg_ref, wl_ref, b_ref, wr_ref, o_ref):
    a = agg_ref[...].astype(jnp.float32)
    a = jnp.where(a == -jnp.inf, 0.0, a)
    z = (jnp.dot(a, wl_ref[...], preferred_element_type=jnp.float32)
         + jnp.dot(x_ref[...], wr_ref[...], preferred_element_type=jnp.float32)
         + b_ref[...])
    col = lax.broadcasted_iota(jnp.int32, z.shape, 1)
    z = jnp.where(col < _C, z, -jnp.inf)
    m = jnp.max(z, axis=1, keepdims=True)
    zs = z - m
    lse = jnp.log(jnp.sum(jnp.exp(zs), axis=1, keepdims=True))
    o_ref[...] = zs - lse


_BN = 400


def _dense_tc(x, agg, wl, b, wr):
    return pl.pallas_call(
        _tc_body,
        grid=(_N // _BN,),
        in_specs=[
            pl.BlockSpec((_BN, _D), lambda i: (i, 0)),
            pl.BlockSpec((_BN, _D), lambda i: (i, 0)),
            pl.BlockSpec((_D, 128), lambda i: (0, 0)),
            pl.BlockSpec((1, 128), lambda i: (0, 0)),
            pl.BlockSpec((_D, 128), lambda i: (0, 0)),
        ],
        out_specs=pl.BlockSpec((_BN, 128), lambda i: (i, 0)),
        out_shape=jax.ShapeDtypeStruct((_N, 128), jnp.float32),
    )(x, agg, wl, b, wr)


def kernel(x, edge_index, W_l, b_l, W_r):
    src = edge_index[0]
    dst = edge_index[1]
    aggf = _segment_max_sc(x, src, dst)
    agg = jax.lax.bitcast_convert_type(
        aggf.reshape(_NPAD, _D2), jnp.bfloat16).reshape(_NPAD, _D)
    wl = jnp.zeros((_D, 128), jnp.float32).at[:, :_C].set(W_l.T)
    wr = jnp.zeros((_D, 128), jnp.float32).at[:, :_C].set(W_r.T)
    b = jnp.zeros((1, 128), jnp.float32).at[0, :_C].set(b_l)
    out = _dense_tc(x, agg, wl, b, wr)
    return out[:, :_C]
